# Initial kernel scaffold; baseline (speedup 1.0000x reference)
import jax
import jax.numpy as jnp
from jax.experimental import pallas as pl


def kernel(x, edge_index, W1, b1, W2, b2, Wout, bout):
    raise NotImplementedError("write your pallas kernel here")


# speedup vs baseline: 15.7785x; 15.7785x over previous
import jax
import jax.numpy as jnp
from jax import lax
from jax.experimental import pallas as pl
from jax.experimental.pallas import tpu as pltpu
from jax.experimental.pallas import tpu_sc as plsc

_N = 10000
_E = 320000
_NPAD = 10240
_NSUB = 16
_EC = _E // _NSUB
_ROWS = (_EC + 127) // 128
_ECP = _ROWS * 128
_REM = _EC - (_ROWS - 1) * 128
_NS = _NPAD // _NSUB


def _rsqrt16(d):
    i = lax.bitcast_convert_type(d, jnp.int32)
    i = jnp.int32(0x5F3759DF) - lax.shift_right_arithmetic(i, 1)
    y = lax.bitcast_convert_type(i, jnp.float32)
    for _ in range(3):
        y = y * (1.5 - 0.5 * d * y * y)
    return y


def _sc_body(src_hbm, dst_hbm, v_hbm, w_hbm,
             src2d, dst2d, vals, cfull, zbuf, abuf, cbuf, sbuf, tbuf,
             acc_a, acc_b, carr, vcarr):
    wid = lax.axis_index("s")
    ebase = wid * _EC
    nbase = wid * _NS
    nsl = pl.ds(nbase, _NS)

    def z16(i, _):
        zbuf[pl.ds(i * 16, 16)] = jnp.zeros((16,), jnp.float32)
        return 0
    lax.fori_loop(0, _NS // 16, z16, 0)
    pltpu.sync_copy(zbuf, acc_a.at[nsl])
    pltpu.sync_copy(zbuf, acc_b.at[nsl])

    def ld(i, _):
        pltpu.sync_copy(src_hbm.at[pl.ds(ebase + i * 128, 128)], src2d.at[i])
        pltpu.sync_copy(dst_hbm.at[pl.ds(ebase + i * 128, 128)], dst2d.at[i])
        return 0
    lax.fori_loop(0, _ROWS - 1, ld, 0)
    last = _ROWS - 1
    pltpu.sync_copy(src_hbm.at[pl.ds(ebase + last * 128, _REM)],
                    src2d.at[last, pl.ds(0, _REM)])
    pltpu.sync_copy(dst_hbm.at[pl.ds(ebase + last * 128, _REM)],
                    dst2d.at[last, pl.ds(0, _REM)])
    pv = jnp.int32(_N) + (wid * 16 + lax.iota(jnp.int32, 16)) % (_NPAD - _N)
    for k in range(_REM // 16, 128 // 16):
        src2d[last, pl.ds(k * 16, 16)] = pv
        dst2d[last, pl.ds(k * 16, 16)] = pv

    def ones16(i, _):
        vals[pl.ds(i * 16, 16)] = jnp.ones((16,), jnp.float32)
        return 0
    lax.fori_loop(0, _EC // 16, ones16, 0)
    def zeros16(i, _):
        vals[pl.ds(i * 16, 16)] = jnp.zeros((16,), jnp.float32)
        return 0
    lax.fori_loop(_EC // 16, _ECP // 16, zeros16, 0)

    plsc.subcore_barrier()

    def hist(j, _):
        vsl = vals.at[pl.ds(j * 128, 128)]
        pltpu.sync_copy(vsl, acc_a.at[src2d.at[j]], add=True)
        pltpu.sync_copy(vsl, acc_b.at[dst2d.at[j]], add=True)
        return 0
    lax.fori_loop(0, _ROWS, hist, 0)
    plsc.subcore_barrier()

    pltpu.sync_copy(acc_a.at[nsl], sbuf)
    def fin_a(i, _):
        s = pl.ds(i * 16, 16)
        abuf[s] = _rsqrt16(jnp.maximum(sbuf[s], 1.0))
        return 0
    lax.fori_loop(0, _NS // 16, fin_a, 0)
    pltpu.sync_copy(acc_b.at[nsl], sbuf)
    def fin_c(i, _):
        s = pl.ds(i * 16, 16)
        cbuf[s] = _rsqrt16(jnp.maximum(sbuf[s], 1.0))
        return 0
    lax.fori_loop(0, _NS // 16, fin_c, 0)
    pltpu.sync_copy(cbuf, carr.at[nsl])
    pltpu.sync_copy(zbuf, acc_a.at[nsl])
    pltpu.sync_copy(zbuf, acc_b.at[nsl])
    plsc.subcore_barrier()

    pltpu.sync_copy(carr, cfull)
    def gouter(j, _):
        def ginner(k, _):
            idx = dst2d[j, pl.ds(k * 16, 16)]
            vals[pl.ds(j * 128 + k * 16, 16)] = plsc.load_gather(cfull, [idx])
            return 0
        lax.fori_loop(0, 8, ginner, 0)
        return 0
    lax.fori_loop(0, _ROWS - 1, gouter, 0)
    for k in range(_REM // 16):
        idx = dst2d[last, pl.ds(k * 16, 16)]
        vals[pl.ds(last * 128 + k * 16, 16)] = plsc.load_gather(cfull, [idx])

    def sc1(j, _):
        pltpu.sync_copy(vals.at[pl.ds(j * 128, 128)],
                        acc_a.at[src2d.at[j]], add=True)
        return 0
    lax.fori_loop(0, _ROWS, sc1, 0)
    plsc.subcore_barrier()

    pltpu.sync_copy(acc_a.at[nsl], sbuf)
    def fin_v(i, _):
        s = pl.ds(i * 16, 16)
        vv = abuf[s] * sbuf[s]
        tbuf[s] = vv
        cbuf[s] = vv * cbuf[s]
        return 0
    lax.fori_loop(0, _NS // 16, fin_v, 0)
    pltpu.sync_copy(tbuf, v_hbm.at[nsl])
    pltpu.sync_copy(cbuf, vcarr.at[nsl])
    plsc.subcore_barrier()

    pltpu.sync_copy(vcarr, cfull)
    lax.fori_loop(0, _ROWS - 1, gouter, 0)
    for k in range(_REM // 16):
        idx = dst2d[last, pl.ds(k * 16, 16)]
        vals[pl.ds(last * 128 + k * 16, 16)] = plsc.load_gather(cfull, [idx])

    def sc2(j, _):
        pltpu.sync_copy(vals.at[pl.ds(j * 128, 128)],
                        acc_b.at[src2d.at[j]], add=True)
        return 0
    lax.fori_loop(0, _ROWS, sc2, 0)
    plsc.subcore_barrier()

    pltpu.sync_copy(acc_b.at[nsl], sbuf)
    def fin_w(i, _):
        s = pl.ds(i * 16, 16)
        tbuf[s] = abuf[s] * sbuf[s]
        return 0
    lax.fori_loop(0, _NS // 16, fin_w, 0)
    pltpu.sync_copy(tbuf, w_hbm.at[nsl])


_sc_fn = pl.kernel(
    _sc_body,
    out_type=(jax.ShapeDtypeStruct((_NPAD,), jnp.float32),
              jax.ShapeDtypeStruct((_NPAD,), jnp.float32)),
    mesh=plsc.VectorSubcoreMesh(core_axis_name="c", subcore_axis_name="s",
                                num_cores=1, num_subcores=_NSUB),
    compiler_params=pltpu.CompilerParams(needs_layout_passes=False),
    scratch_types=[
        pltpu.VMEM((_ROWS, 128), jnp.int32),
        pltpu.VMEM((_ROWS, 128), jnp.int32),
        pltpu.VMEM((_ECP,), jnp.float32),
        pltpu.VMEM((_NPAD,), jnp.float32),
        pltpu.VMEM((_NS,), jnp.float32),
        pltpu.VMEM((_NS,), jnp.float32),
        pltpu.VMEM((_NS,), jnp.float32),
        pltpu.VMEM((_NS,), jnp.float32),
        pltpu.VMEM((_NS,), jnp.float32),
        pltpu.VMEM_SHARED((_NPAD,), jnp.float32),
        pltpu.VMEM_SHARED((_NPAD,), jnp.float32),
        pltpu.VMEM_SHARED((_NPAD,), jnp.float32),
        pltpu.VMEM_SHARED((_NPAD,), jnp.float32),
    ],
)


def _tc_body(x_ref, w_ref, v_ref, w1_ref, b1_ref, w2_ref, b2_ref,
             wo_ref, bo_ref, o_ref):
    wx = jnp.sum(x_ref[...] * w_ref[...], axis=0, keepdims=True)
    sv = jnp.sum(v_ref[...])
    mm = lambda a, b: lax.dot_general(a, b, (((1,), (0,)), ((), ())),
                                      precision=lax.Precision.HIGHEST)
    t1 = mm(wx, w1_ref[...]) + sv * b1_ref[...]
    t2 = mm(t1, w2_ref[...]) + jnp.float32(_N) * b2_ref[...]
    o_ref[...] = mm(t2, wo_ref[...]) + jnp.float32(_N) * bo_ref[...]


_tc_fn = pl.pallas_call(
    _tc_body,
    out_shape=jax.ShapeDtypeStruct((1, 64), jnp.float32),
)


def kernel(x, edge_index, W1, b1, W2, b2, Wout, bout):
    src = edge_index[0]
    dst = edge_index[1]
    v_pad, w_pad = _sc_fn(src, dst)
    out = _tc_fn(x, w_pad[:_N].reshape(_N, 1), v_pad.reshape(_NPAD // 128, 128),
                 W1, b1.reshape(1, -1), W2, b2.reshape(1, -1),
                 Wout, bout.reshape(1, -1))
    return out[0]


# speedup vs baseline: 43.0018x; 2.7253x over previous
import jax
import jax.numpy as jnp
from jax import lax
from jax.experimental import pallas as pl
from jax.experimental.pallas import tpu as pltpu
from jax.experimental.pallas import tpu_sc as plsc

_N = 10000
_E = 320000
_NPAD = 10240
_NSUB = 16
_EC = _E // _NSUB
_ROWS = 160
_ECP = _ROWS * 128
_FULL = _EC // 128
_REM = _EC - _FULL * 128
_NS = _NPAD // _NSUB


def _rsqrt16(d):
    i = lax.bitcast_convert_type(d, jnp.int32)
    i = jnp.int32(0x5F3759DF) - lax.shift_right_arithmetic(i, 1)
    y = lax.bitcast_convert_type(i, jnp.float32)
    for _ in range(3):
        y = y * (1.5 - 0.5 * d * y * y)
    return y


def _scatter_pass(vals, idx2d, acc, sem, per):
    def body(g, _):
        j0 = g * per
        descs = [
            pltpu.async_copy(vals.at[pl.ds((j0 + t) * 128, 128)],
                             acc.at[idx2d.at[j0 + t]], sem, add=True)
            for t in range(per)
        ]
        for d in descs:
            d.wait()
        return 0
    lax.fori_loop(0, _ROWS // per, body, 0)


def _sc_body(src_hbm, dst_hbm, v_hbm, w_hbm,
             src_flat, dst_flat, src2d, dst2d, vals, cfull,
             zbuf, abuf, cbuf, sbuf, tbuf, sem,
             acc_a, acc_b, carr, vcarr):
    wid = lax.axis_index("s")
    ebase = wid * _EC
    nbase = wid * _NS
    nsl = pl.ds(nbase, _NS)

    def z16(i, _):
        zbuf[pl.ds(i * 16, 16)] = jnp.zeros((16,), jnp.float32)
        return 0
    lax.fori_loop(0, _NS // 16, z16, 0)
    pltpu.sync_copy(zbuf, acc_a.at[nsl])
    pltpu.sync_copy(zbuf, acc_b.at[nsl])

    d_src = pltpu.async_copy(src_hbm.at[pl.ds(ebase, _EC)],
                             src_flat.at[pl.ds(0, _EC)], sem)
    d_dst = pltpu.async_copy(dst_hbm.at[pl.ds(ebase, _EC)],
                             dst_flat.at[pl.ds(0, _EC)], sem)
    d_src.wait()
    d_dst.wait()

    def repack(j, _):
        def inner(k, _):
            s = pl.ds(j * 128 + k * 16, 16)
            d = pl.ds(k * 16, 16)
            src2d[j, d] = src_flat[s]
            dst2d[j, d] = dst_flat[s]
            return 0
        lax.fori_loop(0, 8, inner, 0)
        return 0
    lax.fori_loop(0, _FULL, repack, 0)
    pv = jnp.int32(_N) + (wid * 16 + lax.iota(jnp.int32, 16)) % (_NPAD - _N)
    for k in range(_REM // 16):
        s = pl.ds(_FULL * 128 + k * 16, 16)
        src2d[_FULL, pl.ds(k * 16, 16)] = src_flat[s]
        dst2d[_FULL, pl.ds(k * 16, 16)] = dst_flat[s]
    for j in range(_FULL, _ROWS):
        for k in range((_REM // 16) if j == _FULL else 0, 8):
            src2d[j, pl.ds(k * 16, 16)] = pv
            dst2d[j, pl.ds(k * 16, 16)] = pv

    def ones16(i, _):
        vals[pl.ds(i * 16, 16)] = jnp.ones((16,), jnp.float32)
        return 0
    lax.fori_loop(0, _EC // 16, ones16, 0)
    def zeros16(i, _):
        vals[pl.ds(i * 16, 16)] = jnp.zeros((16,), jnp.float32)
        return 0
    lax.fori_loop(_EC // 16, _ECP // 16, zeros16, 0)

    plsc.subcore_barrier()

    def hist(g, _):
        j0 = g * 4
        descs = []
        for t in range(4):
            vsl = vals.at[pl.ds((j0 + t) * 128, 128)]
            descs.append(pltpu.async_copy(vsl, acc_a.at[src2d.at[j0 + t]],
                                          sem, add=True))
            descs.append(pltpu.async_copy(vsl, acc_b.at[dst2d.at[j0 + t]],
                                          sem, add=True))
        for d in descs:
            d.wait()
        return 0
    lax.fori_loop(0, _ROWS // 4, hist, 0)
    plsc.subcore_barrier()

    pltpu.sync_copy(acc_a.at[nsl], sbuf)
    def fin_a(i, _):
        s = pl.ds(i * 16, 16)
        abuf[s] = _rsqrt16(jnp.maximum(sbuf[s], 1.0))
        return 0
    lax.fori_loop(0, _NS // 16, fin_a, 0)
    pltpu.sync_copy(acc_b.at[nsl], sbuf)
    def fin_c(i, _):
        s = pl.ds(i * 16, 16)
        cbuf[s] = _rsqrt16(jnp.maximum(sbuf[s], 1.0))
        return 0
    lax.fori_loop(0, _NS // 16, fin_c, 0)
    pltpu.sync_copy(cbuf, carr.at[nsl])
    pltpu.sync_copy(zbuf, acc_a.at[nsl])
    pltpu.sync_copy(zbuf, acc_b.at[nsl])
    plsc.subcore_barrier()

    pltpu.sync_copy(carr, cfull)
    def gat(i, _):
        s = pl.ds(i * 16, 16)
        vals[s] = plsc.load_gather(cfull, [dst_flat[s]])
        return 0
    lax.fori_loop(0, _EC // 16, gat, 0)
    _scatter_pass(vals, src2d, acc_a, sem, 8)
    plsc.subcore_barrier()

    pltpu.sync_copy(acc_a.at[nsl], sbuf)
    def fin_v(i, _):
        s = pl.ds(i * 16, 16)
        vv = abuf[s] * sbuf[s]
        tbuf[s] = vv
        cbuf[s] = vv * cbuf[s]
        return 0
    lax.fori_loop(0, _NS // 16, fin_v, 0)
    pltpu.sync_copy(tbuf, v_hbm.at[nsl])
    pltpu.sync_copy(cbuf, vcarr.at[nsl])
    plsc.subcore_barrier()

    pltpu.sync_copy(vcarr, cfull)
    lax.fori_loop(0, _EC // 16, gat, 0)
    _scatter_pass(vals, src2d, acc_b, sem, 8)
    plsc.subcore_barrier()

    pltpu.sync_copy(acc_b.at[nsl], sbuf)
    def fin_w(i, _):
        s = pl.ds(i * 16, 16)
        tbuf[s] = abuf[s] * sbuf[s]
        return 0
    lax.fori_loop(0, _NS // 16, fin_w, 0)
    pltpu.sync_copy(tbuf, w_hbm.at[nsl])


_sc_fn = pl.kernel(
    _sc_body,
    out_type=(jax.ShapeDtypeStruct((_NPAD,), jnp.float32),
              jax.ShapeDtypeStruct((_NPAD,), jnp.float32)),
    mesh=plsc.VectorSubcoreMesh(core_axis_name="c", subcore_axis_name="s",
                                num_cores=1, num_subcores=_NSUB),
    compiler_params=pltpu.CompilerParams(needs_layout_passes=False),
    scratch_types=[
        pltpu.VMEM((_ECP,), jnp.int32),
        pltpu.VMEM((_ECP,), jnp.int32),
        pltpu.VMEM((_ROWS, 128), jnp.int32),
        pltpu.VMEM((_ROWS, 128), jnp.int32),
        pltpu.VMEM((_ECP,), jnp.float32),
        pltpu.VMEM((_NPAD,), jnp.float32),
        pltpu.VMEM((_NS,), jnp.float32),
        pltpu.VMEM((_NS,), jnp.float32),
        pltpu.VMEM((_NS,), jnp.float32),
        pltpu.VMEM((_NS,), jnp.float32),
        pltpu.VMEM((_NS,), jnp.float32),
        pltpu.SemaphoreType.DMA,
        pltpu.VMEM_SHARED((_NPAD,), jnp.float32),
        pltpu.VMEM_SHARED((_NPAD,), jnp.float32),
        pltpu.VMEM_SHARED((_NPAD,), jnp.float32),
        pltpu.VMEM_SHARED((_NPAD,), jnp.float32),
    ],
)


def _tc_body(x_ref, w_ref, v_ref, w1_ref, b1_ref, w2_ref, b2_ref,
             wo_ref, bo_ref, o_ref):
    wx = jnp.sum(x_ref[...] * w_ref[...], axis=0, keepdims=True)
    sv = jnp.sum(v_ref[...])
    mm = lambda a, b: lax.dot_general(a, b, (((1,), (0,)), ((), ())),
                                      precision=lax.Precision.HIGHEST)
    t1 = mm(wx, w1_ref[...]) + sv * b1_ref[...]
    t2 = mm(t1, w2_ref[...]) + jnp.float32(_N) * b2_ref[...]
    o_ref[...] = mm(t2, wo_ref[...]) + jnp.float32(_N) * bo_ref[...]


_tc_fn = pl.pallas_call(
    _tc_body,
    out_shape=jax.ShapeDtypeStruct((1, 64), jnp.float32),
)


def kernel(x, edge_index, W1, b1, W2, b2, Wout, bout):
    src = edge_index[0]
    dst = edge_index[1]
    v_pad, w_pad = _sc_fn(src, dst)
    out = _tc_fn(x, w_pad[:_N].reshape(_N, 1), v_pad.reshape(_NPAD // 128, 128),
                 W1, b1.reshape(1, -1), W2, b2.reshape(1, -1),
                 Wout, bout.reshape(1, -1))
    return out[0]
